# baseline (speedup 1.0000x reference)
import functools

import jax
import jax.numpy as jnp
from jax import lax
from jax.experimental import pallas as pl
from jax.experimental.pallas import tpu as pltpu
from jax.experimental.pallas import tpu_sc as plsc

_N = 32 * 512 * 512
_NW = 32
_PER_W = _N // _NW
_CHUNK = 32768
_NCHUNK = _PER_W // _CHUNK
_LANES = 16
_THRES = 3.0

_mesh = plsc.VectorSubcoreMesh(core_axis_name="c", subcore_axis_name="s")


@functools.partial(
    pl.kernel,
    mesh=_mesh,
    out_type=jax.ShapeDtypeStruct((_NW, 2, _LANES), jnp.int32),
    scratch_types=[
        pltpu.VMEM((_CHUNK,), jnp.float32),
        pltpu.VMEM((_CHUNK,), jnp.float32),
        pltpu.VMEM((2, _LANES), jnp.int32),
    ],
)
def _partial_counts(o_hbm, t_hbm, out_hbm, o_buf, t_buf, res_buf):
    cid = lax.axis_index("c")
    sid = lax.axis_index("s")
    wid = sid * 2 + cid
    base = wid * _PER_W

    av = jnp.zeros((_LANES,), jnp.int32)
    ae = jnp.zeros((_LANES,), jnp.int32)
    for i in range(_NCHUNK):
        off = base + i * _CHUNK
        pltpu.sync_copy(o_hbm.at[pl.ds(off, _CHUNK)], o_buf)
        pltpu.sync_copy(t_hbm.at[pl.ds(off, _CHUNK)], t_buf)

        def body(k, carry, o_buf=o_buf, t_buf=t_buf):
            av, ae = carry
            sl = pl.ds(pl.multiple_of(k * _LANES, _LANES), _LANES)
            o = o_buf[sl]
            t = t_buf[sl]
            m = (t > 0.0) & (o > 0.0)
            e = m & (jnp.abs(t - o) > _THRES)
            one = jnp.ones((_LANES,), jnp.int32)
            zero = jnp.zeros((_LANES,), jnp.int32)
            av = av + jnp.where(m, one, zero)
            ae = ae + jnp.where(e, one, zero)
            return av, ae

        av, ae = lax.fori_loop(0, _CHUNK // _LANES, body, (av, ae))

    res_buf[0, :] = av
    res_buf[1, :] = ae
    pltpu.sync_copy(res_buf, out_hbm.at[wid])


def _finalize_body(p_ref, o_ref):
    p = p_ref[...].astype(jnp.float32)
    valid = jnp.sum(p[:, 0, :])
    err = jnp.sum(p[:, 1, :])
    o_ref[0, 0] = err / valid


_finalize = pl.pallas_call(
    _finalize_body,
    out_shape=jax.ShapeDtypeStruct((1, 1), jnp.float32),
    out_specs=pl.BlockSpec(memory_space=pltpu.SMEM),
)


def kernel(outputs, target):
    o = jnp.reshape(outputs, (_N,))
    t = jnp.reshape(target, (_N,))
    parts = _partial_counts(o, t)
    return _finalize(parts)[0, 0]


# speedup vs baseline: 1.1606x; 1.1606x over previous
import functools

import jax
import jax.numpy as jnp
from jax import lax
from jax.experimental import pallas as pl
from jax.experimental.pallas import tpu as pltpu
from jax.experimental.pallas import tpu_sc as plsc

_N = 32 * 512 * 512
_NW = 32
_PER_W = _N // _NW
_CHUNK = 16384
_NCHUNK = _PER_W // _CHUNK
_LANES = 16
_THRES = 3.0
_ERR_BIT = 0x10001

_mesh = plsc.VectorSubcoreMesh(core_axis_name="c", subcore_axis_name="s")


@functools.partial(
    pl.kernel,
    mesh=_mesh,
    out_type=jax.ShapeDtypeStruct((_NW, _LANES), jnp.int32),
    scratch_types=[
        pltpu.VMEM((_CHUNK,), jnp.float32),
        pltpu.VMEM((_CHUNK,), jnp.float32),
        pltpu.VMEM((_CHUNK,), jnp.float32),
        pltpu.VMEM((_CHUNK,), jnp.float32),
        pltpu.VMEM((_LANES,), jnp.int32),
        pltpu.SemaphoreType.DMA,
        pltpu.SemaphoreType.DMA,
        pltpu.SemaphoreType.DMA,
        pltpu.SemaphoreType.DMA,
    ],
)
def _partial_counts(o_hbm, t_hbm, out_hbm, ob0, ob1, tb0, tb1, res_buf,
                    so0, so1, st0, st1):
    wid = lax.axis_index("s") * 2 + lax.axis_index("c")
    base = wid * _PER_W
    obufs, tbufs = (ob0, ob1), (tb0, tb1)
    osems, tsems = (so0, so1), (st0, st1)

    def dmas(slot, chunk):
        off = base + chunk * _CHUNK
        return (
            pltpu.make_async_copy(o_hbm.at[pl.ds(off, _CHUNK)], obufs[slot],
                                  osems[slot]),
            pltpu.make_async_copy(t_hbm.at[pl.ds(off, _CHUNK)], tbufs[slot],
                                  tsems[slot]),
        )

    for d in dmas(0, 0):
        d.start()

    acc = jnp.zeros((_LANES,), jnp.int32)
    for i in range(_NCHUNK):
        s = i & 1
        if i + 1 < _NCHUNK:
            for d in dmas(1 - s, i + 1):
                d.start()
        for d in dmas(s, i):
            d.wait()
        o_buf, t_buf = obufs[s], tbufs[s]

        def body(k, acc, o_buf=o_buf, t_buf=t_buf):
            sl = pl.ds(pl.multiple_of(k, _LANES), _LANES)
            o = o_buf[sl]
            t = t_buf[sl]
            m = jnp.minimum(t, o) > 0.0
            big = jnp.abs(t - o) > _THRES
            inc = jnp.where(m, jnp.where(big, _ERR_BIT, 1), 0)
            return acc + inc

        acc = plsc.parallel_loop(0, _CHUNK, step=_LANES, unroll=8,
                                 carry=acc)(body)

    res_buf[...] = acc
    pltpu.sync_copy(res_buf, out_hbm.at[wid])


def _finalize_body(p_ref, o_ref):
    p = p_ref[...]
    valid = jnp.sum((p & 0xFFFF).astype(jnp.float32))
    err = jnp.sum((p >> 16).astype(jnp.float32))
    o_ref[0, 0] = err / valid


_finalize = pl.pallas_call(
    _finalize_body,
    out_shape=jax.ShapeDtypeStruct((1, 1), jnp.float32),
    out_specs=pl.BlockSpec(memory_space=pltpu.SMEM),
)


def kernel(outputs, target):
    o = jnp.reshape(outputs, (_N,))
    t = jnp.reshape(target, (_N,))
    parts = _partial_counts(o, t)
    return _finalize(parts)[0, 0]


# speedup vs baseline: 1.8207x; 1.5688x over previous
import functools

import jax
import jax.numpy as jnp
from jax import lax
from jax.experimental import pallas as pl
from jax.experimental.pallas import tpu as pltpu
from jax.experimental.pallas import tpu_sc as plsc

_B = 32
_H = 512
_W = 512
_NW = 32
_CH = 32
_NCHUNK = _H // _CH
_LANES = 16
_THRES = 3.0
_ERR_BIT = 0x10001

_mesh = plsc.VectorSubcoreMesh(core_axis_name="c", subcore_axis_name="s")


@functools.partial(
    pl.kernel,
    mesh=_mesh,
    out_type=jax.ShapeDtypeStruct((_NW, _LANES), jnp.int32),
    scratch_types=[
        pltpu.VMEM((_CH, _W), jnp.float32),
        pltpu.VMEM((_CH, _W), jnp.float32),
        pltpu.VMEM((_CH, _W), jnp.float32),
        pltpu.VMEM((_CH, _W), jnp.float32),
        pltpu.VMEM((_LANES,), jnp.int32),
        pltpu.SemaphoreType.DMA,
        pltpu.SemaphoreType.DMA,
        pltpu.SemaphoreType.DMA,
        pltpu.SemaphoreType.DMA,
    ],
)
def _partial_counts(o_hbm, t_hbm, out_hbm, ob0, ob1, tb0, tb1, res_buf,
                    so0, so1, st0, st1):
    wid = lax.axis_index("s") * 2 + lax.axis_index("c")
    obufs, tbufs = (ob0, ob1), (tb0, tb1)
    osems, tsems = (so0, so1), (st0, st1)

    def dmas(slot, chunk):
        r0 = chunk * _CH
        return (
            pltpu.make_async_copy(o_hbm.at[wid, pl.ds(r0, _CH), :],
                                  obufs[slot], osems[slot]),
            pltpu.make_async_copy(t_hbm.at[wid, pl.ds(r0, _CH), :],
                                  tbufs[slot], tsems[slot]),
        )

    for d in dmas(0, 0):
        d.start()

    acc = jnp.zeros((_LANES,), jnp.int32)
    for i in range(_NCHUNK):
        s = i & 1
        if i + 1 < _NCHUNK:
            for d in dmas(1 - s, i + 1):
                d.start()
        for d in dmas(s, i):
            d.wait()
        o_buf, t_buf = obufs[s], tbufs[s]

        def body(k, acc, o_buf=o_buf, t_buf=t_buf):
            r = lax.shift_right_logical(k, 9)
            c = lax.bitwise_and(k, _W - 1)
            sl = pl.ds(pl.multiple_of(c, _LANES), _LANES)
            o = o_buf[r, sl]
            t = t_buf[r, sl]
            m = jnp.minimum(t, o) > 0.0
            big = jnp.abs(t - o) > _THRES
            inc = jnp.where(m, jnp.where(big, _ERR_BIT, 1), 0)
            return acc + inc

        acc = plsc.parallel_loop(0, _CH * _W, step=_LANES, unroll=8,
                                 carry=acc)(body)

    res_buf[...] = acc
    pltpu.sync_copy(res_buf, out_hbm.at[wid])


def _finalize_body(p_ref, o_ref):
    p = p_ref[...]
    valid = jnp.sum((p & 0xFFFF).astype(jnp.float32))
    err = jnp.sum((p >> 16).astype(jnp.float32))
    o_ref[0, 0] = err / valid


_finalize = pl.pallas_call(
    _finalize_body,
    out_shape=jax.ShapeDtypeStruct((1, 1), jnp.float32),
    out_specs=pl.BlockSpec(memory_space=pltpu.SMEM),
)


def kernel(outputs, target):
    parts = _partial_counts(outputs, target)
    return _finalize(parts)[0, 0]


# speedup vs baseline: 3.3840x; 1.8586x over previous
import functools

import jax
import jax.numpy as jnp
from jax import lax
from jax.experimental import pallas as pl
from jax.experimental.pallas import tpu as pltpu
from jax.experimental.pallas import tpu_sc as plsc

_B = 32
_H = 512
_W = 512
_SC_B = 8
_TC_B = _B - _SC_B
_NW = 32
_RPW = _SC_B * _H // _NW
_CH = 32
_NCHUNK = _RPW // _CH
_WPB = _H // _RPW
_LANES = 16
_THRES = 3.0
_ERR_BIT = 0x10001

_mesh = plsc.VectorSubcoreMesh(core_axis_name="c", subcore_axis_name="s")


@functools.partial(
    pl.kernel,
    mesh=_mesh,
    out_type=jax.ShapeDtypeStruct((_NW, _LANES), jnp.int32),
    scratch_types=[
        pltpu.VMEM((_CH, _W), jnp.float32),
        pltpu.VMEM((_CH, _W), jnp.float32),
        pltpu.VMEM((_CH, _W), jnp.float32),
        pltpu.VMEM((_CH, _W), jnp.float32),
        pltpu.VMEM((_LANES,), jnp.int32),
        pltpu.SemaphoreType.DMA,
        pltpu.SemaphoreType.DMA,
        pltpu.SemaphoreType.DMA,
        pltpu.SemaphoreType.DMA,
    ],
)
def _sc_partial(o_hbm, t_hbm, out_hbm, ob0, ob1, tb0, tb1, res_buf,
                so0, so1, st0, st1):
    wid = lax.axis_index("s") * 2 + lax.axis_index("c")
    batch = wid // _WPB
    row0 = (wid % _WPB) * _RPW
    obufs, tbufs = (ob0, ob1), (tb0, tb1)
    osems, tsems = (so0, so1), (st0, st1)

    def dmas(slot, chunk):
        r0 = row0 + chunk * _CH
        return (
            pltpu.make_async_copy(o_hbm.at[batch, pl.ds(r0, _CH), :],
                                  obufs[slot], osems[slot]),
            pltpu.make_async_copy(t_hbm.at[batch, pl.ds(r0, _CH), :],
                                  tbufs[slot], tsems[slot]),
        )

    for d in dmas(0, 0):
        d.start()

    acc = jnp.zeros((_LANES,), jnp.int32)
    for i in range(_NCHUNK):
        s = i & 1
        if i + 1 < _NCHUNK:
            for d in dmas(1 - s, i + 1):
                d.start()
        for d in dmas(s, i):
            d.wait()
        o_buf, t_buf = obufs[s], tbufs[s]

        def row_body(r, acc, o_buf=o_buf, t_buf=t_buf):
            def col_body(c, a):
                incs = []
                for g in range(8):
                    sl = pl.ds(pl.multiple_of(c + g * _LANES, _LANES),
                               _LANES)
                    o = o_buf[r, sl]
                    t = t_buf[r, sl]
                    m = jnp.minimum(t, o) > 0.0
                    big = jnp.abs(t - o) > _THRES
                    incs.append(jnp.where(m, jnp.where(big, _ERR_BIT, 1), 0))
                while len(incs) > 1:
                    incs = [x + y for x, y in zip(incs[::2], incs[1::2])]
                return a + incs[0]

            return plsc.parallel_loop(0, _W, step=8 * _LANES, unroll=2,
                                      carry=acc)(col_body)

        acc = lax.fori_loop(0, _CH, row_body, acc)

    res_buf[...] = acc
    pltpu.sync_copy(res_buf, out_hbm.at[wid])


def _tc_partial_body(o_ref, t_ref, out_ref):
    i = pl.program_id(0)
    o = o_ref[0]
    t = t_ref[0]
    m = jnp.minimum(t, o) > 0.0
    big = jnp.abs(t - o) > _THRES
    valid = jnp.sum(m.astype(jnp.float32))
    err = jnp.sum(jnp.where(m & big, 1.0, 0.0))

    @pl.when(i == 0)
    def _():
        out_ref[0, 0] = 0.0
        out_ref[0, 1] = 0.0

    out_ref[0, 0] += valid
    out_ref[0, 1] += err


_tc_partial = pl.pallas_call(
    _tc_partial_body,
    grid=(_TC_B,),
    in_specs=[
        pl.BlockSpec((1, _H, _W), lambda i: (_SC_B + i, 0, 0)),
        pl.BlockSpec((1, _H, _W), lambda i: (_SC_B + i, 0, 0)),
    ],
    out_specs=pl.BlockSpec((1, 2), lambda i: (0, 0),
                           memory_space=pltpu.SMEM),
    out_shape=jax.ShapeDtypeStruct((1, 2), jnp.float32),
)


def _finalize_body(p_ref, tc_ref, o_ref):
    p = p_ref[...]
    valid = jnp.sum((p & 0xFFFF).astype(jnp.float32)) + tc_ref[0, 0]
    err = jnp.sum((p >> 16).astype(jnp.float32)) + tc_ref[0, 1]
    o_ref[0, 0] = err / valid


_finalize = pl.pallas_call(
    _finalize_body,
    in_specs=[
        pl.BlockSpec(memory_space=pltpu.VMEM),
        pl.BlockSpec(memory_space=pltpu.SMEM),
    ],
    out_shape=jax.ShapeDtypeStruct((1, 1), jnp.float32),
    out_specs=pl.BlockSpec(memory_space=pltpu.SMEM),
)


def kernel(outputs, target):
    sc_parts = _sc_partial(outputs, target)
    tc_parts = _tc_partial(outputs, target)
    return _finalize(sc_parts, tc_parts)[0, 0]
